# baseline (speedup 1.0000x reference)
import jax
import jax.numpy as jnp
from jax.experimental import pallas as pl

_BI = 40


def _body(x_ref, w0t_ref, b0_ref, w1t_ref, b1_ref, a0_ref, a1_ref, out_ref):
    i = pl.program_id(0)
    xb = x_ref[...]
    y0 = (jnp.dot(xb, w0t_ref[...], preferred_element_type=jnp.float32)
          + b0_ref[...]).astype(jnp.bfloat16)
    y1 = (jnp.dot(xb, w1t_ref[...], preferred_element_type=jnp.float32)
          + b1_ref[...]).astype(jnp.bfloat16)
    a0 = a0_ref[...].astype(jnp.bfloat16)
    a1 = a1_ref[...].astype(jnp.bfloat16)
    dn = (((0,), (0,)), ((), ()))
    p0 = jax.lax.dot_general(a0, y0, dn, preferred_element_type=jnp.float32)
    p1 = jax.lax.dot_general(a1, y1, dn, preferred_element_type=jnp.float32)

    @pl.when(i == 0)
    def _init():
        out_ref[...] = p0 + p1

    @pl.when(i > 0)
    def _acc():
        out_ref[...] += p0 + p1


def kernel(A_r0, A_r1, x, W0, b0, W1, b1):
    n, d_in = x.shape
    d_out = W0.shape[0]
    return pl.pallas_call(
        _body,
        grid=(n // _BI,),
        in_specs=[
            pl.BlockSpec((_BI, d_in), lambda i: (i, 0)),
            pl.BlockSpec((d_in, d_out), lambda i: (0, 0)),
            pl.BlockSpec((1, d_out), lambda i: (0, 0)),
            pl.BlockSpec((d_in, d_out), lambda i: (0, 0)),
            pl.BlockSpec((1, d_out), lambda i: (0, 0)),
            pl.BlockSpec((_BI, n), lambda i: (i, 0)),
            pl.BlockSpec((_BI, n), lambda i: (i, 0)),
        ],
        out_specs=pl.BlockSpec((n, d_out), lambda i: (0, 0)),
        out_shape=jax.ShapeDtypeStruct((n, d_out), x.dtype),
    )(x, W0.T, b0[None, :], W1.T, b1[None, :], A_r0, A_r1)


# speedup vs baseline: 3.6537x; 3.6537x over previous
import jax
import jax.numpy as jnp
from jax.experimental import pallas as pl
from jax.experimental.pallas import tpu as pltpu

_BI = 200


def _body(x_ref, w0t_ref, b0_ref, w1t_ref, b1_ref, a0_ref, a1_ref,
          out_ref, acc_ref):
    i = pl.program_id(0)
    ni = pl.num_programs(0)
    xb = x_ref[...]
    y0 = (jnp.dot(xb, w0t_ref[...], preferred_element_type=jnp.float32)
          + b0_ref[...])
    y1 = (jnp.dot(xb, w1t_ref[...], preferred_element_type=jnp.float32)
          + b1_ref[...])
    y0t = y0.T.astype(jnp.bfloat16)
    y1t = y1.T.astype(jnp.bfloat16)
    a0 = a0_ref[...].astype(jnp.bfloat16)
    a1 = a1_ref[...].astype(jnp.bfloat16)
    dn = (((1,), (0,)), ((), ()))
    p0 = jax.lax.dot_general(y0t, a0, dn, preferred_element_type=jnp.float32)
    p1 = jax.lax.dot_general(y1t, a1, dn, preferred_element_type=jnp.float32)

    @pl.when(i == 0)
    def _init():
        acc_ref[...] = p0 + p1

    @pl.when(i > 0)
    def _acc():
        acc_ref[...] += p0 + p1

    @pl.when(i == ni - 1)
    def _finish():
        out_ref[...] = acc_ref[...].T


def kernel(A_r0, A_r1, x, W0, b0, W1, b1):
    n, d_in = x.shape
    d_out = W0.shape[0]
    return pl.pallas_call(
        _body,
        grid=(n // _BI,),
        in_specs=[
            pl.BlockSpec((_BI, d_in), lambda i: (i, 0)),
            pl.BlockSpec((d_in, d_out), lambda i: (0, 0)),
            pl.BlockSpec((1, d_out), lambda i: (0, 0)),
            pl.BlockSpec((d_in, d_out), lambda i: (0, 0)),
            pl.BlockSpec((1, d_out), lambda i: (0, 0)),
            pl.BlockSpec((_BI, n), lambda i: (i, 0)),
            pl.BlockSpec((_BI, n), lambda i: (i, 0)),
        ],
        out_specs=pl.BlockSpec((n, d_out), lambda i: (0, 0)),
        out_shape=jax.ShapeDtypeStruct((n, d_out), x.dtype),
        scratch_shapes=[pltpu.VMEM((d_out, n), jnp.float32)],
    )(x, W0.T, b0[None, :], W1.T, b1[None, :], A_r0, A_r1)


# speedup vs baseline: 3.6784x; 1.0067x over previous
import jax
import jax.numpy as jnp
from jax.experimental import pallas as pl
from jax.experimental.pallas import tpu as pltpu

_BI = 200


def _body(x_ref, w0t_ref, b0_ref, w1t_ref, b1_ref, a0_ref, a1_ref,
          out_ref, acc_ref):
    i = pl.program_id(0)
    ni = pl.num_programs(0)
    xb = x_ref[...]
    dnw = (((1,), (1,)), ((), ()))
    y0 = (jax.lax.dot_general(xb, w0t_ref[...], dnw,
                              preferred_element_type=jnp.float32)
          + b0_ref[...])
    y1 = (jax.lax.dot_general(xb, w1t_ref[...], dnw,
                              preferred_element_type=jnp.float32)
          + b1_ref[...])
    y0t = y0.T.astype(jnp.bfloat16)
    y1t = y1.T.astype(jnp.bfloat16)
    a0 = a0_ref[...].astype(jnp.bfloat16)
    a1 = a1_ref[...].astype(jnp.bfloat16)
    dn = (((1,), (0,)), ((), ()))
    p0 = jax.lax.dot_general(y0t, a0, dn, preferred_element_type=jnp.float32)
    p1 = jax.lax.dot_general(y1t, a1, dn, preferred_element_type=jnp.float32)

    @pl.when(i == 0)
    def _init():
        acc_ref[...] = p0 + p1

    @pl.when(i > 0)
    def _acc():
        acc_ref[...] += p0 + p1

    @pl.when(i == ni - 1)
    def _finish():
        out_ref[...] = acc_ref[...].T


def kernel(A_r0, A_r1, x, W0, b0, W1, b1):
    n, d_in = x.shape
    d_out = W0.shape[0]
    return pl.pallas_call(
        _body,
        grid=(n // _BI,),
        in_specs=[
            pl.BlockSpec((_BI, d_in), lambda i: (i, 0)),
            pl.BlockSpec((d_out, d_in), lambda i: (0, 0)),
            pl.BlockSpec((1, d_out), lambda i: (0, 0)),
            pl.BlockSpec((d_out, d_in), lambda i: (0, 0)),
            pl.BlockSpec((1, d_out), lambda i: (0, 0)),
            pl.BlockSpec((_BI, n), lambda i: (i, 0)),
            pl.BlockSpec((_BI, n), lambda i: (i, 0)),
        ],
        out_specs=pl.BlockSpec((n, d_out), lambda i: (0, 0)),
        out_shape=jax.ShapeDtypeStruct((n, d_out), x.dtype),
        scratch_shapes=[pltpu.VMEM((d_out, n), jnp.float32)],
    )(x, W0, b0[None, :], W1, b1[None, :], A_r0, A_r1)
